# Initial kernel scaffold; baseline (speedup 1.0000x reference)
import jax
import jax.numpy as jnp
from jax.experimental import pallas as pl


def kernel(x, edge_index, batch, params):
    raise NotImplementedError("write your pallas kernel here")


# speedup vs baseline: 14.9233x; 14.9233x over previous
import functools
import math

import jax
import jax.numpy as jnp
from jax import lax
from jax.experimental import pallas as pl
from jax.experimental.pallas import tpu as pltpu
from jax.experimental.pallas import tpu_sc as plsc

F = 128
G = 64
RATIO = 0.5
CH = 256
NEG = float("-inf")
HI = lax.Precision.HIGHEST
DEF = lax.Precision.DEFAULT


def _pad_rows(a, np_, fill=0):
    pad = np_ - a.shape[0]
    return jnp.pad(a, ((0, pad),) + ((0, 0),) * (a.ndim - 1), constant_values=fill)


def _agg_call(x_p, src, dst, zeros128, np_, interpret=False):
    E = src.shape[0]
    NSC, NSUB = 2, 16
    NW = NSC * NSUB
    epc = E // NW
    assert E % NW == 0 and np_ % NSUB == 0
    rpw = np_ // NSUB
    nfull = epc // 128
    rem = epc - nfull * 128
    assert rem % 8 == 0

    def body(x_hbm, src_hbm, dst_hbm, z_hbm, out_hbm,
             sidx, didx, rows, sidx_t, didx_t, rows_t, zbuf, shared, sem):
        cid = lax.axis_index("c")
        sid = lax.axis_index("s")
        wid = cid * NSUB + sid
        row0 = sid * rpw
        pltpu.sync_copy(z_hbm, zbuf)
        nz = rpw // 128
        for t in range(nz):
            pltpu.sync_copy(zbuf, shared.at[pl.ds(row0 + t * 128, 128)])
        rz = rpw - nz * 128
        if rz:
            pltpu.sync_copy(zbuf.at[pl.ds(0, rz)],
                            shared.at[pl.ds(row0 + nz * 128, rz)])
        plsc.subcore_barrier()

        base = wid * epc

        def step(t, carry):
            off = pl.multiple_of(base + t * 128, 8)
            pltpu.sync_copy(src_hbm.at[pl.ds(off, 128)], sidx.at[0])
            pltpu.sync_copy(dst_hbm.at[pl.ds(off, 128)], didx.at[0])
            pltpu.async_copy(x_hbm.at[sidx.at[0]], rows, sem).wait()
            pltpu.sync_copy(rows, shared.at[didx.at[0]], add=True)
            return carry

        lax.fori_loop(0, nfull, step, 0)
        if rem:
            off = pl.multiple_of(base + nfull * 128, 8)
            pltpu.sync_copy(src_hbm.at[pl.ds(off, rem)], sidx_t.at[0])
            pltpu.sync_copy(dst_hbm.at[pl.ds(off, rem)], didx_t.at[0])
            pltpu.async_copy(x_hbm.at[sidx_t.at[0]], rows_t, sem).wait()
            pltpu.sync_copy(rows_t, shared.at[didx_t.at[0]], add=True)
        plsc.subcore_barrier()

        ob = cid * np_ + row0
        for t in range(nz):
            pltpu.sync_copy(shared.at[pl.ds(row0 + t * 128, 128)], zbuf)
            pltpu.sync_copy(zbuf, out_hbm.at[pl.ds(ob + t * 128, 128)])
        if rz:
            pltpu.sync_copy(shared.at[pl.ds(row0 + nz * 128, rz)],
                            zbuf.at[pl.ds(0, rz)])
            pltpu.sync_copy(zbuf.at[pl.ds(0, rz)],
                            out_hbm.at[pl.ds(ob + nz * 128, rz)])

    fn = pl.kernel(
        body,
        out_type=jax.ShapeDtypeStruct((2 * np_, F), jnp.float32),
        mesh=plsc.VectorSubcoreMesh(core_axis_name="c", subcore_axis_name="s",
                                    num_cores=NSC, num_subcores=NSUB),
        scratch_types=[
            pltpu.VMEM((1, 128), jnp.int32),
            pltpu.VMEM((1, 128), jnp.int32),
            pltpu.VMEM((128, F), jnp.float32),
            pltpu.VMEM((1, max(rem, 8)), jnp.int32),
            pltpu.VMEM((1, max(rem, 8)), jnp.int32),
            pltpu.VMEM((max(rem, 8), F), jnp.float32),
            pltpu.VMEM((128, F), jnp.float32),
            pltpu.VMEM_SHARED((np_, F), jnp.float32),
            pltpu.SemaphoreType.DMA,
        ],
        interpret=interpret,
    )
    return fn(x_p, src, dst, zeros128)


def _starts_call(batch_row, np_, interpret=False):
    def body(b_ref, starts_ref, counts_ref):
        br = b_ref[...]
        gcol = lax.broadcasted_iota(jnp.int32, (G, np_), 0)
        onehot = (gcol == br).astype(jnp.float32)
        counts = jnp.sum(onehot, axis=1, keepdims=True)
        lt = (lax.broadcasted_iota(jnp.int32, (G, G), 1)
              < lax.broadcasted_iota(jnp.int32, (G, G), 0)).astype(jnp.float32)
        starts = lax.dot_general(lt, counts, (((1,), (0,)), ((), ())),
                                 precision=HI)
        starts_ref[...] = starts.astype(jnp.int32)
        counts_ref[...] = counts.astype(jnp.int32)

    starts, counts = pl.pallas_call(
        body,
        out_shape=[jax.ShapeDtypeStruct((G, 1), jnp.int32),
                   jax.ShapeDtypeStruct((G, 1), jnp.int32)],
        interpret=interpret,
    )(batch_row)
    return starts.reshape(G), counts.reshape(G)


def _dense_call(x_p, agg0, agg1, nm_col, W1, b1, W2, b2, P1, pb1, P2, pb2,
                w_col, np_, interpret=False):
    BR = 1312
    assert np_ % BR == 0
    grid = np_ // BR

    def body(x_ref, a0_ref, a1_ref, nm_ref, w1_ref, b1_ref, w2_ref, b2_ref,
             p1_ref, q1_ref, p2_ref, q2_ref, w_ref, h_ref, lp_ref, sm_ref):
        xa = x_ref[...] + a0_ref[...] + a1_ref[...]
        t = jnp.maximum(
            lax.dot_general(xa, w1_ref[...], (((1,), (0,)), ((), ())),
                            precision=DEF) + b1_ref[...], 0.0)
        h = jnp.maximum(
            lax.dot_general(t, w2_ref[...], (((1,), (0,)), ((), ())),
                            precision=DEF) + b2_ref[...], 0.0)
        u = jnp.maximum(
            lax.dot_general(h, p1_ref[...], (((1,), (0,)), ((), ())),
                            precision=DEF) + q1_ref[...], 0.0)
        lp = lax.dot_general(u, p2_ref[...], (((1,), (0,)), ((), ())),
                             precision=DEF) + q2_ref[...]
        w = w_ref[...]
        s = lax.dot_general(h, w, (((1,), (0,)), ((), ())),
                            precision=DEF) / jnp.sqrt(jnp.sum(w * w))
        h_ref[...] = h
        lp_ref[...] = lp
        sm_ref[...] = jnp.where(nm_ref[...] > 0, s, NEG)

    row_spec = pl.BlockSpec((BR, F), lambda i: (i, 0))
    col_spec = pl.BlockSpec((BR, 1), lambda i: (i, 0))
    full = lambda shp: pl.BlockSpec(shp, lambda i: (0, 0))
    return pl.pallas_call(
        body,
        grid=(grid,),
        in_specs=[row_spec, row_spec, row_spec, col_spec,
                  full((F, F)), full((1, F)), full((F, F)), full((1, F)),
                  full((F, F)), full((1, F)), full((F, F)), full((1, F)),
                  full((F, 1))],
        out_specs=[row_spec, row_spec, col_spec],
        out_shape=[jax.ShapeDtypeStruct((np_, F), jnp.float32),
                   jax.ShapeDtypeStruct((np_, F), jnp.float32),
                   jax.ShapeDtypeStruct((np_, 1), jnp.float32)],
        interpret=interpret,
    )(x_p, agg0, agg1, nm_col, W1, b1, W2, b2, P1, pb1, P2, pb2, w_col)


def _level_call(starts, counts, h, lp, smask, nm_col, np_, g1=None, g2=None,
                interpret=False):
    last = g1 is not None
    nchunks = np_ // CH

    def body(*refs):
        if last:
            (st_ref, ct_ref, h_ref, lp_ref, sm_ref, nm_ref, g1_ref, g2_ref,
             xn_ref, sel_ref, gout_ref, proj_ref, fin_ref) = refs
        else:
            (st_ref, ct_ref, h_ref, lp_ref, sm_ref, nm_ref,
             xn_ref, sel_ref, gout_ref, proj_ref) = refs

        eye = (lax.broadcasted_iota(jnp.int32, (CH, CH), 0)
               == lax.broadcasted_iota(jnp.int32, (CH, CH), 1)
               ).astype(jnp.float32)
        zf = jnp.zeros((CH, F), jnp.float32)
        zc = jnp.zeros((CH, 1), jnp.float32)
        for c in range(nchunks):
            xn_ref[c * CH:(c + 1) * CH, :] = zf
            sel_ref[c * CH:(c + 1) * CH, :] = zc

        def graph_body(g, _):
            start = st_ref[g]
            cnt = ct_ref[g]
            abase = (start // 8) * 8
            nc = (start - abase + cnt + CH - 1) // CH
            end = start + cnt

            def p1(ci, carry):
                pmax, psum, vcnt = carry
                cb = pl.multiple_of(abase + ci * CH, 8)
                rows = lp_ref[pl.ds(cb, CH), :]
                nmk = nm_ref[pl.ds(cb, CH), :]
                ri = lax.broadcasted_iota(jnp.int32, (CH, 1), 0) + cb
                m = (ri >= start) & (ri < end) & (nmk > 0)
                pmax = jnp.maximum(
                    pmax, jnp.max(jnp.where(m, rows, NEG), axis=0,
                                  keepdims=True))
                psum = psum + jnp.sum(jnp.where(m, rows, 0.0), axis=0,
                                      keepdims=True)
                vcnt = vcnt + jnp.sum(m.astype(jnp.float32), axis=0,
                                      keepdims=True).sum(axis=1, keepdims=True)
                return pmax, psum, vcnt

            pmax, psum, vcnt = lax.fori_loop(
                0, nc, p1,
                (jnp.full((1, F), NEG, jnp.float32),
                 jnp.zeros((1, F), jnp.float32),
                 jnp.zeros((1, 1), jnp.float32)))
            prow = jnp.concatenate(
                [pmax, psum / jnp.maximum(vcnt, 1.0)], axis=1)
            proj_ref[pl.ds(g, 1)] = prow.reshape(1, 1, 2 * F)
            k = jnp.ceil(RATIO * vcnt)

            def p2(ci, carry):
                gmax, gsum, scnt = carry
                cb = pl.multiple_of(abase + ci * CH, 8)
                si = sm_ref[pl.ds(cb, CH), :]
                ri = lax.broadcasted_iota(jnp.int32, (CH, 1), 0) + cb
                inr_i = (ri >= start) & (ri < end)
                nmi = nm_ref[pl.ds(cb, CH), :] > 0

                def pj(cj, rank):
                    jb = pl.multiple_of(abase + cj * CH, 8)
                    sjc = sm_ref[pl.ds(jb, CH), :]
                    vj_col = (nm_ref[pl.ds(jb, CH), :] > 0)
                    sj_fin = jnp.where(vj_col, sjc, 0.0)
                    sjr = lax.dot_general(sj_fin, eye, (((0,), (0,)), ((), ())),
                                          precision=HI)
                    vjr = lax.dot_general(vj_col.astype(jnp.float32), eye,
                                          (((0,), (0,)), ((), ())),
                                          precision=HI)
                    jidx = lax.broadcasted_iota(jnp.int32, (CH, CH), 1) + jb
                    iidx = lax.broadcasted_iota(jnp.int32, (CH, CH), 0) + cb
                    beats = (sjr > si) | ((sjr == si) & (jidx < iidx))
                    cmask = beats & (vjr > 0) & (jidx >= start) & (jidx < end)
                    return rank + jnp.sum(cmask.astype(jnp.float32), axis=1,
                                          keepdims=True)

                rank = lax.fori_loop(0, nc, pj, jnp.zeros((CH, 1), jnp.float32))
                selc = (rank < k) & nmi & inr_i
                hk = h_ref[pl.ds(cb, CH), :]
                xn = jnp.where(selc, hk * jnp.tanh(si), 0.0)
                old = xn_ref[pl.ds(cb, CH), :]
                xn_ref[pl.ds(cb, CH), :] = jnp.where(inr_i, xn, old)
                olds = sel_ref[pl.ds(cb, CH), :]
                sel_ref[pl.ds(cb, CH), :] = jnp.where(
                    inr_i, selc.astype(jnp.float32), olds)
                gmax = jnp.maximum(
                    gmax, jnp.max(jnp.where(selc, xn, NEG), axis=0,
                                  keepdims=True))
                gsum = gsum + jnp.sum(xn, axis=0, keepdims=True)
                scnt = scnt + jnp.sum(selc.astype(jnp.float32), axis=0,
                                      keepdims=True).sum(axis=1, keepdims=True)
                return gmax, gsum, scnt

            gmax, gsum, scnt = lax.fori_loop(
                0, nc, p2,
                (jnp.full((1, F), NEG, jnp.float32),
                 jnp.zeros((1, F), jnp.float32),
                 jnp.zeros((1, 1), jnp.float32)))
            grow = jnp.concatenate(
                [gmax, gsum / jnp.maximum(scnt, 1.0)], axis=1)
            gout_ref[pl.ds(g, 1)] = grow.reshape(1, 1, 2 * F)
            return 0

        lax.fori_loop(0, G, graph_body, 0)
        if last:
            fin_ref[...] = (jnp.maximum(g1_ref[...], 0.0)
                            + jnp.maximum(g2_ref[...], 0.0)
                            + jnp.maximum(gout_ref[...], 0.0))

    smem_spec = pl.BlockSpec(memory_space=pltpu.SMEM)
    in_specs = [smem_spec, smem_spec] + [pl.BlockSpec()] * (4 + 2 * last)
    out_shape = [jax.ShapeDtypeStruct((np_, F), jnp.float32),
                 jax.ShapeDtypeStruct((np_, 1), jnp.float32),
                 jax.ShapeDtypeStruct((G, 1, 2 * F), jnp.float32),
                 jax.ShapeDtypeStruct((G, 1, 2 * F), jnp.float32)]
    if last:
        out_shape.append(jax.ShapeDtypeStruct((G, 1, 2 * F), jnp.float32))
    args = [starts, counts, h, lp, smask, nm_col]
    if last:
        args += [g1.reshape(G, 1, 2 * F), g2.reshape(G, 1, 2 * F)]
    res = pl.pallas_call(
        body, in_specs=in_specs, out_shape=out_shape, interpret=interpret,
    )(*args)
    res = list(res)
    for i in range(2, len(res)):
        res[i] = res[i].reshape(G, 2 * F)
    return tuple(res)


def _run(x, edge_index, batch, params, interpret=False):
    N = x.shape[0]
    np_ = ((N + CH - 1) // CH + 1) * CH
    np_ = ((np_ + 1312 - 1) // 1312) * 1312
    x_p = _pad_rows(x.astype(jnp.float32), np_)
    batch_row = _pad_rows(batch.reshape(N, 1), np_, fill=G).reshape(1, np_)
    nm_col = (jnp.arange(np_, dtype=jnp.int32) < N).astype(
        jnp.float32).reshape(np_, 1)
    src = edge_index[0]
    dst = edge_index[1]
    zeros128 = jnp.zeros((128, F), jnp.float32)

    starts, counts = _starts_call(batch_row, np_, interpret)

    gs, projs = [], []
    x_cur, nm_cur = x_p, nm_col
    final = None
    for lvl, (conv, proj, pw) in enumerate(
            [("conv1", "proj1", "pool1_w"), ("conv2", "proj2", "pool2_w"),
             ("conv3", "proj3", "pool3_w")]):
        aggbuf = _agg_call(x_cur, src, dst, zeros128, np_, interpret)
        agg0, agg1 = aggbuf[:np_], aggbuf[np_:]
        h, lp, smask = _dense_call(
            x_cur, agg0, agg1, nm_cur,
            params[conv + "_W1"], params[conv + "_b1"].reshape(1, F),
            params[conv + "_W2"], params[conv + "_b2"].reshape(1, F),
            params[proj + "_W1"], params[proj + "_b1"].reshape(1, F),
            params[proj + "_W2"], params[proj + "_b2"].reshape(1, F),
            params[pw].reshape(F, 1), np_, interpret)
        if lvl < 2:
            xn, sel, g_l, proj_l = _level_call(
                starts, counts, h, lp, smask, nm_cur, np_, interpret=interpret)
        else:
            xn, sel, g_l, proj_l, final = _level_call(
                starts, counts, h, lp, smask, nm_cur, np_,
                g1=gs[0], g2=gs[1], interpret=interpret)
        gs.append(g_l)
        projs.append(proj_l)
        x_cur, nm_cur = xn, sel

    return (final, gs[0], gs[1], gs[2], projs[0], projs[1], projs[2])


def kernel(x, edge_index, batch, params):
    return _run(x, edge_index, batch, params)
